# baseline (speedup 1.0000x reference)
import functools

import jax
import jax.numpy as jnp
from jax import lax
from jax.experimental import pallas as pl
from jax.experimental.pallas import tpu as pltpu
from jax.experimental.pallas import tpu_sc as plsc

_CHUNK = 128


def _build(B, V, D, NC, NS):
    NW = NC * NS
    b_per_w = B // NW
    n_chunks = b_per_w // _CHUNK
    mesh = plsc.VectorSubcoreMesh(core_axis_name="c", subcore_axis_name="s")

    @functools.partial(
        pl.kernel,
        mesh=mesh,
        out_type=jax.ShapeDtypeStruct((B, D), jnp.float32),
        scratch_types=[
            pltpu.VMEM((n_chunks, _CHUNK), jnp.int32),
            pltpu.VMEM((b_per_w, D), jnp.float32),
            pltpu.SemaphoreType.DMA,
        ],
        compiler_params=pltpu.CompilerParams(use_tc_tiling_on_sc=False),
    )
    def gather_kernel(table_hbm, idx_hbm, out_hbm, idx_v, rows_v, sem):
        wid = lax.axis_index("s") * NC + lax.axis_index("c")
        pltpu.sync_copy(idx_hbm.at[pl.ds(wid * n_chunks, n_chunks)], idx_v)
        copies = []
        for j in range(n_chunks):
            copies.append(
                pltpu.async_copy(
                    table_hbm.at[idx_v.at[j]],
                    rows_v.at[pl.ds(j * _CHUNK, _CHUNK)],
                    sem,
                )
            )
        for c in copies:
            c.wait()
        pltpu.sync_copy(rows_v, out_hbm.at[pl.ds(wid * b_per_w, b_per_w)])

    return gather_kernel


def kernel(cls, embedding):
    (B,) = cls.shape
    V, D = embedding.shape
    info = plsc.get_sparse_core_info()
    NC, NS = info.num_cores, info.num_subcores
    idx2d = cls.reshape(-1, _CHUNK)
    return _build(B, V, D, NC, NS)(embedding, idx2d)


# speedup vs baseline: 2.3362x; 2.3362x over previous
import functools

import jax
import jax.numpy as jnp
from jax import lax
from jax.experimental import pallas as pl
from jax.experimental.pallas import tpu as pltpu
from jax.experimental.pallas import tpu_sc as plsc

_BS = 8


def _build(B, V, D, NC, NS):
    NW = NC * NS
    b_per_w = B // NW
    n_blocks = b_per_w // _BS
    mesh = plsc.VectorSubcoreMesh(core_axis_name="c", subcore_axis_name="s")

    @functools.partial(
        pl.kernel,
        mesh=mesh,
        out_type=jax.ShapeDtypeStruct((B, D), jnp.float32),
        scratch_types=[
            pltpu.VMEM((b_per_w + 16,), jnp.int32),
            pltpu.VMEM((_BS, D, 128), jnp.float32),
            pltpu.VMEM((_BS, D), jnp.float32),
            pltpu.SemaphoreType.DMA,
        ],
        compiler_params=pltpu.CompilerParams(needs_layout_passes=False),
    )
    def gather_kernel(tab_hbm, idx_hbm, out_hbm, idx_v, slots, rows, sem):
        wid = lax.axis_index("s") * NC + lax.axis_index("c")
        base = wid * b_per_w
        pltpu.sync_copy(idx_hbm.at[pl.ds(base, b_per_w)],
                        idx_v.at[pl.ds(0, b_per_w)])
        lanes = lax.iota(jnp.int32, 16)

        def block(g, carry):
            ivec = idx_v[pl.ds(g * _BS, 16)]
            rs = [jnp.sum(jnp.where(lanes == l, ivec, 0))
                  for l in range(_BS)]
            copies = []
            for l in range(_BS):
                s = pl.multiple_of(lax.bitwise_and(rs[l], -128), 128)
                copies.append(
                    pltpu.async_copy(
                        tab_hbm.at[:, pl.ds(s, 128)], slots.at[l], sem
                    )
                )
            for c in copies:
                c.wait()
            for l in range(_BS):
                col = jnp.full((16,), lax.bitwise_and(rs[l], 127), jnp.int32)
                for k in range(D // 16):
                    vals = plsc.load_gather(
                        slots.at[l], [lanes + 16 * k, col]
                    )
                    rows[l, pl.ds(16 * k, 16)] = vals
            pltpu.sync_copy(rows, out_hbm.at[pl.ds(base + g * _BS, _BS)])
            return carry

        lax.fori_loop(0, n_blocks, block, 0)

    return gather_kernel


def kernel(cls, embedding):
    (B,) = cls.shape
    V, D = embedding.shape
    info = plsc.get_sparse_core_info()
    NC, NS = info.num_cores, info.num_subcores
    return _build(B, V, D, NC, NS)(embedding.T, cls)


# speedup vs baseline: 2.4459x; 1.0469x over previous
import functools

import jax
import jax.numpy as jnp
from jax import lax
from jax.experimental import pallas as pl
from jax.experimental.pallas import tpu as pltpu
from jax.experimental.pallas import tpu_sc as plsc

_BS = 4


def _build(B, V, D, NC, NS):
    NW = NC * NS
    b_per_w = B // NW
    n_blocks = b_per_w // _BS
    mesh = plsc.VectorSubcoreMesh(core_axis_name="c", subcore_axis_name="s")

    @functools.partial(
        pl.kernel,
        mesh=mesh,
        out_type=jax.ShapeDtypeStruct((B, D), jnp.float32),
        scratch_types=[
            pltpu.VMEM((b_per_w + 16,), jnp.int32),
            pltpu.VMEM((2 * _BS, D, 128), jnp.float32),
            pltpu.VMEM((2 * _BS, D), jnp.float32),
            pltpu.SemaphoreType.DMA,
            pltpu.SemaphoreType.DMA,
        ],
        compiler_params=pltpu.CompilerParams(needs_layout_passes=False),
    )
    def gather_kernel(tab_hbm, idx_hbm, out_hbm, idx_v, slots, rows,
                      sem0, sem1):
        wid = lax.axis_index("s") * NC + lax.axis_index("c")
        base = wid * b_per_w
        pltpu.sync_copy(idx_hbm.at[pl.ds(base, b_per_w)],
                        idx_v.at[pl.ds(0, b_per_w)])
        lanes = lax.iota(jnp.int32, 16)

        def scalars(b):
            off = pl.multiple_of((b // 4) * 16, 16)
            ivec = idx_v[pl.ds(off, 16)]
            lo = (b % 4) * 4
            return [jnp.sum(jnp.where(lanes == lo + l, ivec, 0))
                    for l in range(_BS)]

        def issue(b, bank, sem):
            rs = scalars(b)
            for l in range(_BS):
                s = pl.multiple_of(lax.bitwise_and(rs[l], -128), 128)
                pltpu.async_copy(tab_hbm.at[:, pl.ds(s, 128)],
                                 slots.at[bank * _BS + l], sem)

        def drain(bank, sem):
            for l in range(_BS):
                pltpu.make_async_copy(tab_hbm.at[:, pl.ds(0, 128)],
                                      slots.at[bank * _BS + l], sem).wait()

        def extract(b, bank):
            rs = scalars(b)
            for l in range(_BS):
                col = jnp.full((16,), lax.bitwise_and(rs[l], 127), jnp.int32)
                for k in range(D // 16):
                    vals = plsc.load_gather(
                        slots.at[bank * _BS + l], [lanes + 16 * k, col]
                    )
                    rows[bank * _BS + l, pl.ds(16 * k, 16)] = vals

        issue(0, 0, sem0)

        def body(G, carry):
            b0 = 2 * G
            issue(b0 + 1, 1, sem1)
            drain(0, sem0)
            extract(b0, 0)

            @pl.when(b0 + 2 < n_blocks)
            def _():
                issue(b0 + 2, 0, sem0)

            drain(1, sem1)
            extract(b0 + 1, 1)
            pltpu.sync_copy(rows,
                            out_hbm.at[pl.ds(base + G * 2 * _BS, 2 * _BS)])
            return carry

        lax.fori_loop(0, n_blocks // 2, body, 0)

    return gather_kernel


def kernel(cls, embedding):
    (B,) = cls.shape
    V, D = embedding.shape
    info = plsc.get_sparse_core_info()
    NC, NS = info.num_cores, info.num_subcores
    return _build(B, V, D, NC, NS)(embedding.T, cls)


# speedup vs baseline: 3.5579x; 1.4546x over previous
import functools

import jax
import jax.numpy as jnp
from jax import lax
from jax.experimental import pallas as pl
from jax.experimental.pallas import tpu as pltpu
from jax.experimental.pallas import tpu_sc as plsc

_BS = 4
_CHUNK = 128


def _build(B, V, D, NC, NS):
    NW = NC * NS
    b_per_w = B // NW
    n_blocks = b_per_w // _BS
    n_chunks = b_per_w // _CHUNK
    blocks_per_chunk = _CHUNK // _BS
    pairs_per_chunk = blocks_per_chunk // 2
    mesh = plsc.VectorSubcoreMesh(core_axis_name="c", subcore_axis_name="s")

    @functools.partial(
        pl.kernel,
        mesh=mesh,
        out_type=jax.ShapeDtypeStruct((B, 128), jnp.float32),
        scratch_types=[
            pltpu.VMEM((b_per_w,), jnp.int32),
            pltpu.VMEM((n_chunks, _CHUNK), jnp.int32),
            pltpu.VMEM((2 * _BS, D, 128), jnp.float32),
            pltpu.VMEM((2, _CHUNK, 128), jnp.float32),
            pltpu.SemaphoreType.DMA,
            pltpu.SemaphoreType.DMA,
            pltpu.SemaphoreType.DMA,
        ],
        compiler_params=pltpu.CompilerParams(needs_layout_passes=False),
    )
    def gather_kernel(tab_hbm, sr_hbm, perm_hbm, out_hbm, idx_v, perm_v,
                      slots, rows, sem0, sem1, sem_sc):
        wid = lax.axis_index("s") * NC + lax.axis_index("c")
        base = wid * b_per_w
        pltpu.sync_copy(sr_hbm.at[pl.ds(base, b_per_w)], idx_v)
        pltpu.sync_copy(perm_hbm.at[pl.ds(wid * n_chunks, n_chunks)], perm_v)
        lanes = lax.iota(jnp.int32, 16)

        def scalars(b):
            off = pl.multiple_of((b // 4) * 16, 16)
            ivec = idx_v[pl.ds(off, 16)]
            lo = (b % 4) * 4
            return [jnp.sum(jnp.where(lanes == lo + l, ivec, 0))
                    for l in range(_BS)]

        def fetch_plan(rs):
            ss = [lax.bitwise_and(r, -128) for r in rs]
            nf = [jnp.int32(1)]
            flags = [None]
            for l in range(1, _BS):
                f = ss[l] != ss[l - 1]
                flags.append(f)
                nf.append(nf[l - 1] + jnp.where(f, 1, 0))
            return ss, flags, nf

        def issue(b, bank, sem):
            rs = scalars(b)
            ss, flags, nf = fetch_plan(rs)
            s0 = pl.multiple_of(ss[0], 128)
            pltpu.async_copy(tab_hbm.at[:, pl.ds(s0, 128)],
                             slots.at[bank * _BS], sem)
            for l in range(1, _BS):
                @pl.when(flags[l])
                def _(l=l):
                    s = pl.multiple_of(ss[l], 128)
                    pltpu.async_copy(tab_hbm.at[:, pl.ds(s, 128)],
                                     slots.at[bank * _BS + nf[l] - 1], sem)

        def drain(b, bank, sem):
            rs = scalars(b)
            _, _, nf = fetch_plan(rs)
            dummy = tab_hbm.at[:, pl.ds(0, 128)]
            pltpu.make_async_copy(dummy, slots.at[bank * _BS], sem).wait()
            for k in range(1, _BS):
                @pl.when(nf[_BS - 1] > k)
                def _(k=k):
                    pltpu.make_async_copy(
                        dummy, slots.at[bank * _BS + k], sem).wait()

        def extract(b, bank):
            rs = scalars(b)
            _, _, nf = fetch_plan(rs)
            sb = (b // blocks_per_chunk) % 2
            for l in range(_BS):
                slot = bank * _BS + nf[l] - 1
                col = jnp.full((16,), lax.bitwise_and(rs[l], 127), jnp.int32)
                row = (b % blocks_per_chunk) * _BS + l
                for k in range(D // 16):
                    vals = plsc.load_gather(
                        slots.at[slot], [lanes + 16 * k, col]
                    )
                    rows[sb, row, pl.ds(16 * k, 16)] = vals

        issue(0, 0, sem0)

        def body(G, carry):
            b0 = 2 * G

            @pl.when(jnp.logical_and(G % pairs_per_chunk == 0, G >= 32))
            def _():
                pltpu.make_async_copy(out_hbm.at[pl.ds(0, _CHUNK)],
                                      rows.at[0], sem_sc).wait()

            issue(b0 + 1, 1, sem1)
            drain(b0, 0, sem0)
            extract(b0, 0)

            @pl.when(b0 + 2 < n_blocks)
            def _():
                issue(b0 + 2, 0, sem0)

            drain(b0 + 1, 1, sem1)
            extract(b0 + 1, 1)

            @pl.when(G % pairs_per_chunk == pairs_per_chunk - 1)
            def _():
                c = G // pairs_per_chunk
                sb = c % 2
                pltpu.async_copy(rows.at[sb], out_hbm.at[perm_v.at[c]],
                                 sem_sc)
            return carry

        lax.fori_loop(0, n_blocks // 2, body, 0)
        pltpu.make_async_copy(out_hbm.at[pl.ds(0, _CHUNK)],
                              rows.at[0], sem_sc).wait()
        pltpu.make_async_copy(out_hbm.at[pl.ds(0, _CHUNK)],
                              rows.at[1], sem_sc).wait()

    return gather_kernel


def kernel(cls, embedding):
    (B,) = cls.shape
    V, D = embedding.shape
    info = plsc.get_sparse_core_info()
    NC, NS = info.num_cores, info.num_subcores
    sr, perm = lax.sort_key_val(cls, lax.iota(jnp.int32, B))
    out128 = _build(B, V, D, NC, NS)(embedding.T, sr,
                                     perm.reshape(B // 128, 128))
    return out128[:, :D]
